# baseline (speedup 1.0000x reference)
import jax
import jax.numpy as jnp
from jax.experimental import pallas as pl
from jax.experimental.pallas import tpu as pltpu

_P = 68
_NROW = _P * _P
_PAD = 72
_NPADROW = _NROW + 2 * _PAD


def _fused_kernel(o0_ref, o1_ref, o2_ref, o3_ref, o4_ref,
                  w1_ref, w2fc_ref, fcb_ref, out_ref, y1_ref, a_ref):
    b = pl.program_id(0)
    d = pl.program_id(1)
    o_refs = (o0_ref, o1_ref, o2_ref, o3_ref, o4_ref)

    @pl.when(jnp.logical_and(b == 0, d == 0))
    def _init():
        y1_ref[...] = jnp.zeros_like(y1_ref)

    def _conv1(dz):
        acc = None
        for a in (-1, 0, 1):
            for bb in (-1, 0, 1):
                for cc in (-1, 0, 1):
                    tap = ((a + 1) * 3 + (bb + 1)) * 3 + (cc + 1)
                    s = _P * bb + cc
                    osl = o_refs[dz + a + 2][0, 0, pl.ds(_PAD + s, _NROW), :]
                    t = osl.astype(jnp.float32) * w1_ref[tap, :][None, :]
                    acc = t if acc is None else acc + t
        mask = o_refs[dz + 2][0, 0, _PAD:_PAD + _NROW, :].astype(jnp.float32)
        y1_ref[(d + dz) % 3, _PAD:_PAD + _NROW, :] = acc * mask

    @pl.when(d == 0)
    def _warmup():
        _conv1(0)
        y1_ref[2, _PAD:_PAD + _NROW, :] = jnp.zeros((_NROW, 16), jnp.float32)
    _conv1(1)

    for dz in (-1, 0, 1):
        for bb in (-1, 0, 1):
            for cc in (-1, 0, 1):
                g = ((dz + 1) * 3 + (bb + 1)) * 3 + (cc + 1)
                s = _P * bb + cc
                a_ref[:, 16 * g:16 * g + 16] = y1_ref[
                    (d + dz) % 3, pl.ds(_PAD + s, _NROW), :].astype(jnp.bfloat16)

    u = jnp.dot(a_ref[:, :], w2fc_ref[:, :],
                preferred_element_type=jnp.float32)
    mask10 = o2_ref[0, 0, _PAD:_PAD + _NROW, 0:10].astype(jnp.float32)
    out_ref[0, 0, :, :] = (u + fcb_ref[0, :][None, :]) * mask10


def kernel(occ, w1, w2, fc_w, fc_b):
    B, D, H, W = occ.shape
    o = occ.astype(jnp.bfloat16)
    o_pad = jnp.pad(o, ((0, 0), (2, 2), (2, 2), (2, 2)))
    o_flat = o_pad.reshape(B, D + 4, _NROW, 1)
    o_flat = jnp.pad(o_flat, ((0, 0), (0, 0), (_PAD, _PAD), (0, 0)))
    o_rep = jnp.broadcast_to(o_flat, (B, D + 4, _NPADROW, 16))
    w1r = w1.reshape(27, 16)
    w2fc = (w2.reshape(432, 32) @ fc_w.T).astype(jnp.bfloat16)
    fcb = fc_b.reshape(1, 10)

    oblk = (1, 1, _NPADROW, 16)
    out = pl.pallas_call(
        _fused_kernel,
        grid=(B, D),
        in_specs=[
            pl.BlockSpec(oblk, lambda b, d: (b, d, 0, 0)),
            pl.BlockSpec(oblk, lambda b, d: (b, d + 1, 0, 0)),
            pl.BlockSpec(oblk, lambda b, d: (b, d + 2, 0, 0)),
            pl.BlockSpec(oblk, lambda b, d: (b, d + 3, 0, 0)),
            pl.BlockSpec(oblk, lambda b, d: (b, d + 4, 0, 0)),
            pl.BlockSpec(w1r.shape, lambda b, d: (0, 0)),
            pl.BlockSpec(w2fc.shape, lambda b, d: (0, 0)),
            pl.BlockSpec(fcb.shape, lambda b, d: (0, 0)),
        ],
        out_specs=pl.BlockSpec((1, 1, _NROW, 10), lambda b, d: (b, d, 0, 0)),
        out_shape=jax.ShapeDtypeStruct((B, D, _NROW, 10), jnp.float32),
        scratch_shapes=[
            pltpu.VMEM((3, _NPADROW, 16), jnp.float32),
            pltpu.VMEM((_NROW, 432), jnp.bfloat16),
        ],
    )(o_rep, o_rep, o_rep, o_rep, o_rep, w1r, w2fc, fcb)
    out = out.reshape(B, D, _P, _P, 10)[:, :, 2:2 + H, 2:2 + W, :]
    return out.reshape(B * D * H * W, 10)


# speedup vs baseline: 9.8855x; 9.8855x over previous
import jax
import jax.numpy as jnp
from jax.experimental import pallas as pl
from jax.experimental.pallas import tpu as pltpu

_P = 68
_NROW = _P * _P
_PAD = 72
_NPADROW = _NROW + 2 * _PAD
_ZB = 6
_NS = 11


def kernel(occ, w1, w2, fc_w, fc_b):
    B, D, H, W = occ.shape
    o = occ.astype(jnp.float32)
    opl = jnp.pad(o, ((0, 0), (2, 4), (2, 2), (2, 2))).reshape(B, D + 6, _NROW)
    opl = jnp.pad(opl, ((0, 0), (0, 0), (_PAD, _PAD)))
    idx = jnp.arange(_NS)[:, None] * _ZB + jnp.arange(10)[None, :]
    o_win = opl[:, idx, :]
    o_win = jnp.moveaxis(o_win, 2, 3)
    o_win = jnp.broadcast_to(o_win[..., None], (B, _NS, _NPADROW, 10, 16))
    o_win = o_win.reshape(B, _NS, _NPADROW, 160)

    w2fc = (w2.reshape(432, 32) @ fc_w.T).reshape(3, 9, 16, 10)
    rearr = jnp.transpose(w2fc, (0, 2, 1, 3))
    cols = [jnp.pad(rearr, ((q, 5 - q), (0, 0), (0, 0), (0, 6)))
            for q in range(_ZB)]
    b6 = jnp.stack(cols, axis=3)
    b6 = jnp.pad(b6, ((0, 0), (0, 0), (0, 0), (0, 2), (0, 0)))
    b6 = b6.reshape(128, 9 * 128).astype(jnp.bfloat16)

    fcb6 = jnp.pad(fc_b, (0, 6))
    fcb6 = jnp.tile(fcb6, 8).reshape(1, 128)

    w1r = w1.reshape(27, 16)
    w1tile = jnp.tile(w1r, (1, 8))

    def body(owin_ref, b6_ref, fcb6_ref, w1t_ref, out_ref,
             os1_ref, os2_ref, ysum_ref, ycat_ref, u_ref):
        @pl.when(jnp.logical_and(pl.program_id(0) == 0,
                                 pl.program_id(1) == 0))
        def _init():
            ycat_ref[...] = jnp.zeros_like(ycat_ref)

        os1_ref[...] = owin_ref[0, 0, :, 16:144]
        os2_ref[...] = owin_ref[0, 0, :, 32:160]

        def _pass(reader, a):
            terms = None
            for bb in (-1, 0, 1):
                for cc in (-1, 0, 1):
                    tap = ((a + 1) * 3 + (bb + 1)) * 3 + (cc + 1)
                    sft = _P * bb + cc
                    t = reader(sft) * w1t_ref[tap, :][None, :]
                    terms = t if terms is None else terms + t
            return terms

        ysum_ref[_PAD:_PAD + _NROW, :] = _pass(
            lambda sft: owin_ref[0, 0, pl.ds(_PAD + sft, _NROW), 0:128], -1)
        ysum_ref[_PAD:_PAD + _NROW, :] += _pass(
            lambda sft: os1_ref[pl.ds(_PAD + sft, _NROW), :], 0)
        t2 = _pass(lambda sft: os2_ref[pl.ds(_PAD + sft, _NROW), :], 1)
        ycat_ref[_PAD:_PAD + _NROW, :] = (
            (ysum_ref[_PAD:_PAD + _NROW, :] + t2)
            * os1_ref[_PAD:_PAD + _NROW, :]).astype(jnp.bfloat16)

        u_ref[...] = jnp.dot(ycat_ref[...], b6_ref[...],
                             preferred_element_type=jnp.float32)

        acc = None
        for bb in (-1, 0, 1):
            for cc in (-1, 0, 1):
                g = (bb + 1) * 3 + (cc + 1)
                sft = _P * bb + cc
                t = u_ref[pl.ds(_PAD + sft, _NROW), 128 * g:128 * g + 128]
                acc = t if acc is None else acc + t
        full = ((acc + fcb6_ref[0, :][None, :])
                * os2_ref[_PAD:_PAD + _NROW, :])
        out_ref[0, 0, :, :] = full[:, 0:96]

    out = pl.pallas_call(
        body,
        grid=(B, _NS),
        in_specs=[
            pl.BlockSpec((1, 1, _NPADROW, 160), lambda b, st: (b, st, 0, 0)),
            pl.BlockSpec(b6.shape, lambda b, st: (0, 0)),
            pl.BlockSpec(fcb6.shape, lambda b, st: (0, 0)),
            pl.BlockSpec(w1tile.shape, lambda b, st: (0, 0)),
        ],
        out_specs=pl.BlockSpec((1, 1, _NROW, 96), lambda b, st: (b, st, 0, 0)),
        out_shape=jax.ShapeDtypeStruct((B, _NS, _NROW, 96), jnp.float32),
        scratch_shapes=[
            pltpu.VMEM((_NPADROW, 128), jnp.float32),
            pltpu.VMEM((_NPADROW, 128), jnp.float32),
            pltpu.VMEM((_NPADROW, 128), jnp.float32),
            pltpu.VMEM((_NPADROW, 128), jnp.bfloat16),
            pltpu.VMEM((_NPADROW, 9 * 128), jnp.float32),
        ],
    )(o_win, b6, fcb6, w1tile)

    out = out.reshape(B, _NS, _NROW, _ZB, 16)
    out = jnp.moveaxis(out, 3, 2).reshape(B, _NS * _ZB, _NROW, 16)
    out = out[:, :D, :, :10].reshape(B, D, _P, _P, 10)
    out = out[:, :, 2:2 + H, 2:2 + W, :]
    return out.reshape(B * D * H * W, 10)
